# baseline (speedup 1.0000x reference)
import functools

import jax
import jax.numpy as jnp
from jax import lax
from jax.experimental import pallas as pl
from jax.experimental.pallas import tpu as pltpu
from jax.experimental.pallas import tpu_sc as plsc

_U_DIM = 100000
_Z_DIM = 64
_BATCH = 16384

_NC = 2
_NS = 16
_NW = _NC * _NS
_Q = 32768
_NQ = 4
_TAIL = _U_DIM - 3 * _Q
_TAIL_PAD = 1792
_UC = 4096
_UNROLL = 8

_mesh = plsc.VectorSubcoreMesh(core_axis_name="c", subcore_axis_name="s")


@functools.partial(
    pl.kernel,
    mesh=_mesh,
    compiler_params=pltpu.CompilerParams(needs_layout_passes=False),
    out_type=(
        jax.ShapeDtypeStruct((_Z_DIM, _BATCH), jnp.float32),
        jax.ShapeDtypeStruct((_Z_DIM, _BATCH), jnp.float32),
    ),
    scratch_types=[
        pltpu.VMEM((_UC,), jnp.int32),
        pltpu.VMEM((_UC,), jnp.int32),
        pltpu.VMEM((_BATCH + 64,), jnp.int32),
        pltpu.VMEM((_Q,), jnp.float32),
        pltpu.VMEM((_Q,), jnp.float32),
        pltpu.VMEM((_BATCH + 16,), jnp.float32),
        pltpu.SemaphoreType.DMA,
        pltpu.SemaphoreType.DMA,
        pltpu.SemaphoreType.DMA,
        pltpu.SemaphoreType.DMA,
        pltpu.SemaphoreType.DMA,
    ],
)
def _tgather(mt, lt, mtail, ltail, u_hbm, om, ol,
             uc0, uc1, plist, qb0, qb1, outr,
             semu0, semu1, semq0, semq1, osem):
    wid = lax.axis_index("s") * _NC + lax.axis_index("c")
    qsems = (semq0, semq1)
    qbufs = (qb0, qb1)
    ucs = (uc0, uc1)
    usems = (semu0, semu1)
    d0 = wid * 2
    rows = [
        (mt, mtail, om, d0),
        (mt, mtail, om, d0 + 1),
        (lt, ltail, ol, d0),
        (lt, ltail, ol, d0 + 1),
    ]
    iota = jax.lax.broadcasted_iota(jnp.int32, (16,), 0)
    nchunks = _BATCH // _UC

    def u_chunk_copy(cb):
        return pltpu.async_copy(
            u_hbm.at[pl.ds(cb * _UC, _UC)], ucs[cb % 2], usems[cb % 2]
        )

    counts = [jnp.int32(0)] * _NQ
    ucp = [None] * (nchunks + 1)
    ucp[0] = u_chunk_copy(0)
    for cb in range(nchunks):
        if cb + 1 < nchunks:
            ucp[cb + 1] = u_chunk_copy(cb + 1)
        ucp[cb].wait()
        uc = ucs[cb % 2]

        def cbody(i, carry):
            u = uc[pl.ds(i * 16, 16)]
            q = u >> 15
            return tuple(
                carry[j] + jnp.sum((q == j).astype(jnp.int32))
                for j in range(_NQ)
            )

        counts = lax.fori_loop(0, _UC // 16, cbody, tuple(counts))
    c = list(counts)

    def ceil16(x):
        return (x + 15) & ~15

    offs = [jnp.int32(0)] * _NQ
    for j in range(1, _NQ):
        offs[j] = offs[j - 1] + ceil16(c[j - 1])

    ucp[0] = u_chunk_copy(0)
    ptrs = tuple(offs)
    for cb in range(nchunks):
        if cb + 1 < nchunks:
            ucp[cb + 1] = u_chunk_copy(cb + 1)
        ucp[cb].wait()
        uc = ucs[cb % 2]
        cbase = cb * _UC

        def abody(i, carry):
            ptr = carry
            u = uc[pl.ds(i * 16, 16)]
            q = u >> 15
            packed = ((cbase + i * 16) << 15) | (iota << 15) | (u & (_Q - 1))
            new = []
            for j in range(_NQ):
                m = q == j
                plsc.store_compressed(plist.at[pl.ds(ptr[j], 16)], packed, mask=m)
                new.append(ptr[j] + jnp.sum(m.astype(jnp.int32)))
            return tuple(new)

        ptrs = lax.fori_loop(0, _UC // 16, abody, ptrs)

    for j in range(_NQ):
        padlen = (offs[j] + ceil16(c[j])) - ptrs[j]
        dummy = jnp.full((16,), _BATCH << 15, dtype=jnp.int32)
        plsc.store_scatter(plist, [ptrs[j] + iota], dummy, mask=iota < padlen)

    def start_q(r, q):
        src, tail_src, _, d = rows[r]
        if q < _NQ - 1:
            return pltpu.async_copy(
                src.at[d, pl.ds(q * _Q, _Q)],
                qbufs[q % 2].at[pl.ds(0, _Q)],
                qsems[q % 2],
            )
        return pltpu.async_copy(
            tail_src.at[d],
            qbufs[q % 2].at[pl.ds(0, _TAIL_PAD)],
            qsems[q % 2],
        )

    def sweep(q):
        qbuf = qbufs[q % 2]
        base = offs[q]
        nv = ceil16(c[q]) >> 4

        def gbody(i, carry):
            packed = plist[pl.ds(base + i * 16, 16)]
            local = packed & (_Q - 1)
            pos = lax.shift_right_logical(packed, 15)
            val = plsc.load_gather(qbuf, [local])
            plsc.store_scatter(outr, [pos], val)
            return carry

        lax.fori_loop(0, nv, gbody, 0)

    tasks = [(r, q) for r in range(4) for q in range(_NQ)]
    out_pending = None
    cps = [None] * len(tasks)
    cps[0] = start_q(*tasks[0])
    cps[1] = start_q(*tasks[1])
    for k, (r, q) in enumerate(tasks):
        cps[k].wait()
        if q == 0 and out_pending is not None:
            out_pending.wait()
            out_pending = None
        sweep(q)
        if k + 2 < len(tasks):
            cps[k + 2] = start_q(*tasks[k + 2])
        if q == _NQ - 1:
            _, _, dst, d = rows[r]
            out_pending = pltpu.async_copy(
                outr.at[pl.ds(0, _BATCH)], dst.at[d], osem
            )
    out_pending.wait()


def kernel(u, embed_mean, embed_log_var):
    mt = embed_mean.T
    lt = embed_log_var.T
    pad = ((0, 0), (0, _TAIL_PAD - _TAIL))
    mtail = jnp.pad(mt[:, 3 * _Q:], pad)
    ltail = jnp.pad(lt[:, 3 * _Q:], pad)
    om, ol = _tgather(mt, lt, mtail, ltail, u.astype(jnp.int32))
    return om.T, ol.T


# speedup vs baseline: 1.2418x; 1.2418x over previous
import functools

import jax
import jax.numpy as jnp
from jax import lax
from jax.experimental import pallas as pl
from jax.experimental.pallas import tpu as pltpu
from jax.experimental.pallas import tpu_sc as plsc

_U_DIM = 100000
_Z_DIM = 64
_BATCH = 16384

_NC = 2
_NS = 16
_NW = _NC * _NS
_H = 65536
_H1_MAIN = ((_U_DIM - _H) // 128) * 128
_TAIL = _U_DIM - _H - _H1_MAIN
_TAIL_PAD = 128
_H1_BUF = _H1_MAIN + _TAIL_PAD
_UC = 4096
_UNROLL = 8

_mesh = plsc.VectorSubcoreMesh(core_axis_name="c", subcore_axis_name="s")


@functools.partial(
    pl.kernel,
    mesh=_mesh,
    compiler_params=pltpu.CompilerParams(needs_layout_passes=False),
    out_type=(
        jax.ShapeDtypeStruct((_Z_DIM, _BATCH), jnp.float32),
        jax.ShapeDtypeStruct((_Z_DIM, _BATCH), jnp.float32),
    ),
    scratch_types=[
        pltpu.VMEM((_UC,), jnp.int32),
        pltpu.VMEM((_UC,), jnp.int32),
        pltpu.VMEM((_H,), jnp.float32),
        pltpu.VMEM((_H1_BUF,), jnp.float32),
        pltpu.VMEM((_BATCH,), jnp.float32),
        pltpu.SemaphoreType.DMA,
        pltpu.SemaphoreType.DMA,
        pltpu.SemaphoreType.DMA,
        pltpu.SemaphoreType.DMA,
        pltpu.SemaphoreType.DMA,
    ],
)
def _tgather(mt, lt, mtail, ltail, u_hbm, om, ol,
             uc0, uc1, hb0, hb1, outr,
             semu0, semu1, semq0, semq1, osem):
    wid = lax.axis_index("s") * _NC + lax.axis_index("c")
    qsems = (semq0, semq1)
    hbufs = (hb0, hb1)
    ucs = (uc0, uc1)
    usems = (semu0, semu1)
    d0 = wid * 2
    rows = [
        (mt, mtail, om, d0),
        (mt, mtail, om, d0 + 1),
        (lt, ltail, ol, d0),
        (lt, ltail, ol, d0 + 1),
    ]
    iota = jax.lax.broadcasted_iota(jnp.int32, (16,), 0)
    nuc = _BATCH // _UC

    def start_h(r, h):
        src, tail_src, _, d = rows[r]
        if h == 0:
            return [
                pltpu.async_copy(
                    src.at[d, pl.ds(0, _H)], hb0, qsems[0]
                )
            ]
        return [
            pltpu.async_copy(
                src.at[d, pl.ds(_H, _H1_MAIN)],
                hb1.at[pl.ds(0, _H1_MAIN)],
                qsems[1],
            ),
            pltpu.async_copy(
                tail_src.at[d],
                hb1.at[pl.ds(_H1_MAIN, _TAIL_PAD)],
                qsems[1],
            ),
        ]

    def u_chunk_copy(cb):
        return pltpu.async_copy(
            u_hbm.at[pl.ds(cb * _UC, _UC)], ucs[cb % 2], usems[cb % 2]
        )

    def sweep(h):
        hbuf = hbufs[h]
        ucp = [None] * (nuc + 1)
        ucp[0] = u_chunk_copy(0)
        for cb in range(nuc):
            if cb + 1 < nuc:
                ucp[cb + 1] = u_chunk_copy(cb + 1)
            ucp[cb].wait()
            uc = ucs[cb % 2]
            pbase = cb * _UC

            @plsc.parallel_loop(0, _UC, step=16 * _UNROLL)
            def body(i):
                for j in range(_UNROLL):
                    o = i + j * 16
                    u = uc[pl.ds(o, 16)]
                    m = (u >> 16) == h
                    if h == 0:
                        local = u & (_H - 1)
                    else:
                        local = jnp.maximum(u - _H, 0)
                    val = plsc.load_gather(hbuf, [local])
                    plsc.store_scatter(
                        outr, [pbase + o + iota], val, mask=m
                    )

    tasks = [(r, h) for r in range(4) for h in range(2)]
    out_pending = None
    cps = [None] * len(tasks)
    cps[0] = start_h(*tasks[0])
    cps[1] = start_h(*tasks[1])
    for k, (r, h) in enumerate(tasks):
        for cp in cps[k]:
            cp.wait()
        if h == 0 and out_pending is not None:
            out_pending.wait()
            out_pending = None
        sweep(h)
        if k + 2 < len(tasks):
            cps[k + 2] = start_h(*tasks[k + 2])
        if h == 1:
            _, _, dst, d = rows[r]
            out_pending = pltpu.async_copy(outr, dst.at[d], osem)
    out_pending.wait()


def kernel(u, embed_mean, embed_log_var):
    mt = embed_mean.T
    lt = embed_log_var.T
    pad = ((0, 0), (0, _TAIL_PAD - _TAIL))
    mtail = jnp.pad(mt[:, _H + _H1_MAIN:], pad)
    ltail = jnp.pad(lt[:, _H + _H1_MAIN:], pad)
    om, ol = _tgather(mt, lt, mtail, ltail, u.astype(jnp.int32))
    return om.T, ol.T


# speedup vs baseline: 1.5191x; 1.2233x over previous
import functools

import jax
import jax.numpy as jnp
from jax import lax
from jax.experimental import pallas as pl
from jax.experimental.pallas import tpu as pltpu
from jax.experimental.pallas import tpu_sc as plsc

_U_DIM = 100000
_Z_DIM = 64
_BATCH = 16384

_NC = 2
_NS = 16
_NW = _NC * _NS
_CHUNK = 4096
_UNROLL = 16

_mesh = plsc.VectorSubcoreMesh(core_axis_name="c", subcore_axis_name="s")


@functools.partial(
    pl.kernel,
    mesh=_mesh,
    compiler_params=pltpu.CompilerParams(needs_layout_passes=False),
    out_type=(
        jax.ShapeDtypeStruct((_Z_DIM, _BATCH), jnp.float32),
        jax.ShapeDtypeStruct((_Z_DIM, _BATCH), jnp.float32),
    ),
    scratch_types=[
        pltpu.VMEM((_BATCH,), jnp.int32),
        pltpu.VMEM((_U_DIM,), jnp.float32),
        pltpu.VMEM((2, _CHUNK), jnp.float32),
        pltpu.SemaphoreType.DMA,
        pltpu.SemaphoreType.DMA,
    ],
)
def _tgather(mt, lt, u_hbm, om, ol, u_v, row_v, out_v, osem0, osem1):
    wid = lax.axis_index("s") * _NC + lax.axis_index("c")
    pltpu.sync_copy(u_hbm, u_v)
    osems = (osem0, osem1)
    pending = [None, None]
    for k in range(2):
        d = wid * 2 + k
        for src, dst in ((mt, om), (lt, ol)):
            pltpu.sync_copy(src.at[d], row_v)
            for c in range(_BATCH // _CHUNK):
                buf = c % 2
                if pending[buf] is not None:
                    pending[buf].wait()
                    pending[buf] = None

                @plsc.parallel_loop(0, _CHUNK, step=16 * _UNROLL)
                def body(i):
                    for j in range(_UNROLL):
                        idx = u_v[pl.ds(c * _CHUNK + i + j * 16, 16)]
                        out_v[buf, pl.ds(i + j * 16, 16)] = plsc.load_gather(
                            row_v, [idx]
                        )

                pending[buf] = pltpu.async_copy(
                    out_v.at[buf], dst.at[d, pl.ds(c * _CHUNK, _CHUNK)],
                    osems[buf],
                )
    for buf in range(2):
        if pending[buf] is not None:
            pending[buf].wait()


def kernel(u, embed_mean, embed_log_var):
    om, ol = _tgather(embed_mean.T, embed_log_var.T, u.astype(jnp.int32))
    return om.T, ol.T
